# baseline (speedup 1.0000x reference)
import functools

import jax
import jax.numpy as jnp
from jax.experimental import pallas as pl
from jax.experimental.pallas import tpu as pltpu

_NEG_INF = float("-inf")
_BIG_I32 = 2**31 - 1


def _shift(x, s):
    n = x.shape[1]
    return pltpu.roll(x, (n - s) % n, 1)


def _lexgt(av, ai, bv, bi):
    return (av > bv) | ((av == bv) & (ai < bi))


def _merge_sorted_topk(av, ai, rv, ri, k):
    c = _lexgt(rv, ri, av, ai)
    cv = jnp.where(c, rv, av)
    ci = jnp.where(c, ri, ai)
    pos = jax.lax.broadcasted_iota(jnp.int32, cv.shape, 1)
    d = k // 2
    while d >= 1:
        uv, ui = _shift(cv, d), _shift(ci, d)
        dv, di = _shift(cv, -d), _shift(ci, -d)
        upper = (pos % (2 * d)) < d
        take_up = _lexgt(uv, ui, cv, ci)
        take_dn = _lexgt(cv, ci, dv, di)
        swap = (upper & take_up) | (~upper & take_dn)
        ov = jnp.where(upper, uv, dv)
        oi = jnp.where(upper, ui, di)
        cv = jnp.where(swap, ov, cv)
        ci = jnp.where(swap, oi, ci)
        d //= 2
    return cv, ci


def _knn_block(inp_ref, data_ref, out_ref, rv_ref, ri_ref, wk_ref,
               bv_ref, bi_ref, *, nblocks, bn, n, k, num_each, ncls):
    i = pl.program_id(0)

    @pl.when(i == 0)
    def _init():
        rv_ref[:, :] = jnp.full(rv_ref.shape, _NEG_INF, jnp.float32)
        ri_ref[:, :] = jnp.full(ri_ref.shape, jnp.int32(2**30), jnp.int32)

    x = inp_ref[:, :]
    d = data_ref[:, :]
    dot2 = jax.lax.dot_general(
        x * -2.0, d, (((1,), (0,)), ((), ())),
        preferred_element_type=jnp.float32)
    sq_q = jnp.sum(x * x, axis=1, keepdims=True)
    sq_d = jnp.sum(d * d, axis=0, keepdims=True)
    dist = (sq_q + dot2) + sq_d

    start = i * bn
    lane = jax.lax.broadcasted_iota(
        jnp.int32, dist.shape, 1).astype(jnp.float32)

    dist = jnp.where(lane < (n - start), dist, _NEG_INF)
    wk_ref[:, :] = dist

    thr = rv_ref[:, k - 1:k]
    cnt = jnp.sum((dist > thr).astype(jnp.int32), axis=1, keepdims=True)
    need = jnp.max(jnp.minimum(cnt, k))

    @pl.when(need > 0)
    def _reset_block_topk():
        bv_ref[:, :] = jnp.full(bv_ref.shape, _NEG_INF, jnp.float32)
        bi_ref[:, :] = jnp.full(bi_ref.shape, jnp.int32(2**30), jnp.int32)

    for jg in range(0, k, 2):
        @pl.when(jg < need)
        def _extract(jg=jg):
            work = wk_ref[:, :]
            for j in (jg, jg + 1):
                m = jnp.max(work, axis=1, keepdims=True)
                wl = jnp.min(jnp.where(work == m, lane, float(bn)), axis=1,
                             keepdims=True)
                bv_ref[:, k - 1 - j:k - j] = m
                bi_ref[:, k - 1 - j:k - j] = wl.astype(jnp.int32) + start
                work = jnp.where(lane == wl, _NEG_INF, work)
            if jg + 2 < k:
                wk_ref[:, :] = work

    @pl.when(need > 0)
    def _merge():
        nv, ni = _merge_sorted_topk(rv_ref[:, :], ri_ref[:, :],
                                    bv_ref[:, :], bi_ref[:, :], k)
        rv_ref[:, :] = nv
        ri_ref[:, :] = ni

    @pl.when(i == nblocks - 1)
    def _vote():
        cls = ri_ref[:, :] // num_each
        best_cnt = jnp.full((cls.shape[0], 1), -1, jnp.int32)
        best_c = jnp.zeros((cls.shape[0], 1), jnp.int32)
        for c in range(ncls):
            cnt = jnp.sum((cls == c).astype(jnp.int32), axis=1, keepdims=True)
            take = cnt > best_cnt
            best_cnt = jnp.where(take, cnt, best_cnt)
            best_c = jnp.where(take, c, best_c)
        out_ref[:, :] = jnp.broadcast_to(
            best_c.astype(jnp.float32), out_ref.shape)


def _make_call(q, dim, n, k, bn):
    nblocks = -(-n // bn)
    num_each = n // 10
    body = functools.partial(_knn_block, nblocks=nblocks, bn=bn, n=n, k=k,
                             num_each=num_each, ncls=10)
    return pl.pallas_call(
        body,
        grid=(nblocks,),
        in_specs=[
            pl.BlockSpec((q, dim), lambda i: (0, 0)),
            pl.BlockSpec((dim, bn), lambda i: (0, i)),
        ],
        out_specs=pl.BlockSpec((q, 128), lambda i: (0, 0)),
        out_shape=jax.ShapeDtypeStruct((q, 128), jnp.float32),
        scratch_shapes=[
            pltpu.VMEM((q, k), jnp.float32),
            pltpu.VMEM((q, k), jnp.int32),
            pltpu.VMEM((q, bn), jnp.float32),
            pltpu.VMEM((q, k), jnp.float32),
            pltpu.VMEM((q, k), jnp.int32),
        ],
    )


def kernel(inputs, data, k):
    q, dim = inputs.shape
    n = data.shape[0]
    kk = dim
    bn = 1792
    nblocks = -(-n // bn)
    pad = nblocks * bn - n
    data_t = data.T
    if pad:
        data_t = jnp.pad(data_t, ((0, 0), (0, pad)))
    out = _make_call(q, dim, n, kk, bn)(inputs, data_t)
    return out[:, 0]


# speedup vs baseline: 1.9475x; 1.9475x over previous
import functools

import jax
import jax.numpy as jnp
from jax.experimental import pallas as pl
from jax.experimental.pallas import tpu as pltpu

_NEG_INF = float("-inf")
_BIG_I32 = 2**31 - 1


def _shift(x, s):
    n = x.shape[1]
    return pltpu.roll(x, (n - s) % n, 1)


def _lexgt(av, ai, bv, bi):
    return (av > bv) | ((av == bv) & (ai < bi))


def _merge_sorted_topk(av, ai, rv, ri, k):
    c = _lexgt(rv, ri, av, ai)
    cv = jnp.where(c, rv, av)
    ci = jnp.where(c, ri, ai)
    pos = jax.lax.broadcasted_iota(jnp.int32, cv.shape, 1)
    d = k // 2
    while d >= 1:
        uv, ui = _shift(cv, d), _shift(ci, d)
        dv, di = _shift(cv, -d), _shift(ci, -d)
        upper = (pos % (2 * d)) < d
        take_up = _lexgt(uv, ui, cv, ci)
        take_dn = _lexgt(cv, ci, dv, di)
        swap = (upper & take_up) | (~upper & take_dn)
        ov = jnp.where(upper, uv, dv)
        oi = jnp.where(upper, ui, di)
        cv = jnp.where(swap, ov, cv)
        ci = jnp.where(swap, oi, ci)
        d //= 2
    return cv, ci


def _knn_block(inp_ref, data_ref, out_ref, rv_ref, ri_ref, wk_ref,
               bv_ref, bi_ref, *, nblocks, bn, n, k, num_each, ncls):
    i = pl.program_id(0)

    @pl.when(i == 0)
    def _init():
        rv_ref[:, :] = jnp.full(rv_ref.shape, _NEG_INF, jnp.float32)
        ri_ref[:, :] = jnp.full(ri_ref.shape, jnp.int32(2**30), jnp.int32)

    x = inp_ref[:, :]
    d = data_ref[:, :]
    dot2 = jax.lax.dot_general(
        x * -2.0, d, (((1,), (0,)), ((), ())),
        preferred_element_type=jnp.float32)
    sq_q = jnp.sum(x * x, axis=1, keepdims=True)
    sq_d = jnp.sum(d * d, axis=0, keepdims=True)
    dist = (sq_q + dot2) + sq_d

    start = i * bn
    lane = jax.lax.broadcasted_iota(
        jnp.int32, dist.shape, 1).astype(jnp.float32)

    dist = jnp.where(lane < (n - start), dist, _NEG_INF)
    wk_ref[:, :] = dist

    thr = rv_ref[:, k - 1:k]
    cnt = jnp.sum((dist > thr).astype(jnp.int32), axis=1, keepdims=True)
    need = jnp.max(jnp.minimum(cnt, k))

    @pl.when(need > 0)
    def _reset_block_topk():
        bv_ref[:, :] = jnp.full(bv_ref.shape, _NEG_INF, jnp.float32)
        bi_ref[:, :] = jnp.full(bi_ref.shape, jnp.int32(2**30), jnp.int32)

    for jg in range(0, k, 2):
        @pl.when(jg < need)
        def _extract(jg=jg):
            work = wk_ref[:, :]
            for j in (jg, jg + 1):
                m = jnp.max(work, axis=1, keepdims=True)
                wl = jnp.min(jnp.where(work == m, lane, float(bn)), axis=1,
                             keepdims=True)
                bv_ref[:, k - 1 - j:k - j] = m
                bi_ref[:, k - 1 - j:k - j] = wl.astype(jnp.int32) + start
                work = jnp.where(lane == wl, _NEG_INF, work)
            if jg + 2 < k:
                wk_ref[:, :] = work

    @pl.when(need > 0)
    def _merge():
        nv, ni = _merge_sorted_topk(rv_ref[:, :], ri_ref[:, :],
                                    bv_ref[:, :], bi_ref[:, :], k)
        rv_ref[:, :] = nv
        ri_ref[:, :] = ni

    @pl.when(i == nblocks - 1)
    def _vote():
        cls = ri_ref[:, :] // num_each
        best_cnt = jnp.full((cls.shape[0], 1), -1, jnp.int32)
        best_c = jnp.zeros((cls.shape[0], 1), jnp.int32)
        for c in range(ncls):
            cnt = jnp.sum((cls == c).astype(jnp.int32), axis=1, keepdims=True)
            take = cnt > best_cnt
            best_cnt = jnp.where(take, cnt, best_cnt)
            best_c = jnp.where(take, c, best_c)
        out_ref[:, :] = jnp.broadcast_to(
            best_c.astype(jnp.float32), out_ref.shape)


def _make_call(q, dim, n, k, bn):
    nblocks = -(-n // bn)
    num_each = n // 10
    body = functools.partial(_knn_block, nblocks=nblocks, bn=bn, n=n, k=k,
                             num_each=num_each, ncls=10)
    return pl.pallas_call(
        body,
        grid=(nblocks,),
        in_specs=[
            pl.BlockSpec((q, dim), lambda i: (0, 0)),
            pl.BlockSpec((dim, bn), lambda i: (0, i)),
        ],
        out_specs=pl.BlockSpec((q, 128), lambda i: (0, 0)),
        out_shape=jax.ShapeDtypeStruct((q, 128), jnp.float32),
        scratch_shapes=[
            pltpu.VMEM((q, k), jnp.float32),
            pltpu.VMEM((q, k), jnp.int32),
            pltpu.VMEM((q, bn), jnp.float32),
            pltpu.VMEM((q, k), jnp.float32),
            pltpu.VMEM((q, k), jnp.int32),
        ],
    )


def kernel(inputs, data, k):
    q, dim = inputs.shape
    n = data.shape[0]
    kk = dim
    bn = 1664
    nblocks = -(-n // bn)
    pad = nblocks * bn - n
    data_t = data.T
    if pad:
        data_t = jnp.pad(data_t, ((0, 0), (0, pad)))
    out = _make_call(q, dim, n, kk, bn)(inputs, data_t)
    return out[:, 0]
